# baseline (speedup 1.0000x reference)
import jax
import jax.numpy as jnp
from jax.experimental import pallas as pl
from jax.experimental.pallas import tpu as pltpu


def _make_body(n, BI):
    def body(x_ref, row_ref, col_ref, w1_ref, b1_ref, w2_ref, b2_ref,
             o_ref, s1_ref, g_ref, t_ref):
        k = pl.program_id(0)
        s = pl.program_id(1)

        @pl.when(jnp.logical_and(k == 0, s == 0))
        def _():
            s1_ref[...] = jnp.dot(x_ref[...], w1_ref[...],
                                  preferred_element_type=jnp.float32)
            o_ref[...] = jnp.broadcast_to(b2_ref[...], o_ref.shape)

        row_tile = row_ref[...].reshape(BI, BI)

        @pl.when(k < n)
        def _():
            contrib = jnp.dot(row_tile, s1_ref[pl.ds(s * BI, BI), :],
                              preferred_element_type=jnp.float32)

            @pl.when(s == 0)
            def _():
                t_ref[...] = contrib

            @pl.when(s > 0)
            def _():
                t_ref[...] += contrib

            @pl.when(s == n - 1)
            def _():
                h = jnp.maximum(t_ref[...] + b1_ref[...], 0.0)
                g_ref[pl.ds(k * BI, BI), :] = jnp.dot(
                    h, w2_ref[...], preferred_element_type=jnp.float32)

            @pl.when(s < k)
            def _():
                o_ref[pl.ds(k * BI, BI), :] += jnp.dot(
                    row_tile, g_ref[pl.ds(s * BI, BI), :],
                    preferred_element_type=jnp.float32)

        @pl.when(jnp.logical_and(k >= 1, s <= k - 1))
        def _():
            col_tile = col_ref[...].reshape(BI, BI)
            o_ref[pl.ds(s * BI, BI), :] += jnp.dot(
                col_tile, g_ref[pl.ds((k - 1) * BI, BI), :],
                preferred_element_type=jnp.float32)

    return body


def kernel(x, adj, W1, b1, W2, b2):
    N, F = x.shape
    H = W1.shape[1]
    C = W2.shape[1]

    n = 10
    BI = N // n
    assert N == n * BI and BI % 8 == 0

    adj5 = adj.reshape(n, BI, n, 1, BI)
    b1r = b1.reshape(1, H)
    b2r = b2.reshape(1, C)

    def row_map(k, s):
        kk = jnp.minimum(k, n - 1)
        ss = jnp.where(k == n, n - 1, s)
        return (kk, 0, ss, 0, 0)

    def col_map(k, s):
        c = jnp.maximum(k - 1, 0)
        r = jnp.where(k == 0, 0, jnp.minimum(s, k - 1))
        return (r, 0, c, 0, 0)

    out = pl.pallas_call(
        _make_body(n, BI),
        grid=(n + 1, n),
        in_specs=[
            pl.BlockSpec((N, F), lambda k, s: (0, 0)),
            pl.BlockSpec((1, BI, 1, 1, BI), row_map),
            pl.BlockSpec((1, BI, 1, 1, BI), col_map),
            pl.BlockSpec((F, H), lambda k, s: (0, 0)),
            pl.BlockSpec((1, H), lambda k, s: (0, 0)),
            pl.BlockSpec((H, C), lambda k, s: (0, 0)),
            pl.BlockSpec((1, C), lambda k, s: (0, 0)),
        ],
        out_specs=pl.BlockSpec((N, C), lambda k, s: (0, 0)),
        out_shape=jax.ShapeDtypeStruct((N, C), jnp.float32),
        scratch_shapes=[
            pltpu.VMEM((N, H), jnp.float32),
            pltpu.VMEM((N, C), jnp.float32),
            pltpu.VMEM((BI, H), jnp.float32),
        ],
        compiler_params=pltpu.CompilerParams(
            dimension_semantics=("arbitrary", "arbitrary"),
        ),
    )(x, adj5, adj5, W1, b1r, W2, b2r)

    return out


# speedup vs baseline: 6.3260x; 6.3260x over previous
import jax
import jax.numpy as jnp
from jax.experimental import pallas as pl
from jax.experimental.pallas import tpu as pltpu


def _make_body(n_r, n_c, BI, BK, N, TAIL):
    def body(x_ref, row_ref, col_ref, w1_ref, b1_ref, w2_ref, b2_ref,
             o_ref, s1_ref, g_ref, t_ref):
        k = pl.program_id(0)
        s = pl.program_id(1)

        @pl.when(jnp.logical_and(k == 0, s == 0))
        def _():
            s1_ref[pl.ds(0, N), :] = jnp.dot(
                x_ref[...], w1_ref[...], preferred_element_type=jnp.float32)
            o_ref[...] = jnp.broadcast_to(b2_ref[...], o_ref.shape)
            if TAIL > 0:
                s1_ref[pl.ds(N, n_c * BK - N), :] = jnp.zeros(
                    (n_c * BK - N, s1_ref.shape[1]), jnp.float32)
                g_ref[pl.ds(N, n_c * BK - N), :] = jnp.zeros(
                    (n_c * BK - N, g_ref.shape[1]), jnp.float32)

        if TAIL > 0:
            @pl.when(jnp.logical_and(k <= n_r - 1, s == n_c - 1))
            def _():
                row_ref[:, TAIL:] = jnp.zeros(
                    (BI, BK - TAIL), jnp.float32)

            @pl.when(k == n_c + 1)
            def _():
                col_ref[:, TAIL:] = jnp.zeros(
                    (BI, BK - TAIL), jnp.float32)

        @pl.when(jnp.logical_and(k <= n_r - 1, s <= n_c - 1))
        def _():
            tile = row_ref[...]
            contrib = jnp.dot(tile, s1_ref[pl.ds(s * BK, BK), :],
                              preferred_element_type=jnp.float32)

            @pl.when(s == 0)
            def _():
                t_ref[...] = contrib

            @pl.when(s > 0)
            def _():
                t_ref[...] += contrib

            @pl.when(s == n_c - 1)
            def _():
                h = jnp.maximum(t_ref[...] + b1_ref[...], 0.0)
                g_ref[pl.ds(k * BI, BI), :] = jnp.dot(
                    h, w2_ref[...], preferred_element_type=jnp.float32)

            rmax_s = jnp.minimum((BK * (s + 1) - 1) // BI, n_r - 1)

            @pl.when(rmax_s < k)
            def _():
                o_ref[pl.ds(k * BI, BI), :] += jnp.dot(
                    tile, g_ref[pl.ds(s * BK, BK), :],
                    preferred_element_type=jnp.float32)

        c = jnp.minimum(jnp.maximum(k - 2, 0), n_c - 1)
        rmax_c = jnp.minimum((BK * (c + 1) - 1) // BI, n_r - 1)

        @pl.when(jnp.logical_and(k >= 2, s <= rmax_c))
        def _():
            o_ref[pl.ds(s * BI, BI), :] += jnp.dot(
                col_ref[...], g_ref[pl.ds(c * BK, BK), :],
                preferred_element_type=jnp.float32)

    return body


def kernel(x, adj, W1, b1, W2, b2):
    N, F = x.shape
    H = W1.shape[1]
    C = W2.shape[1]

    n_r = 10
    BI = N // n_r
    assert N == n_r * BI and BI % 8 == 0
    BK = 1024 if N >= 10000 else 128
    n_c = -(-N // BK)
    TAIL = N - (n_c - 1) * BK
    if TAIL == BK:
        TAIL = 0
    for c in range(n_c):
        assert min((BK * (c + 1) - 1) // BI, n_r - 1) <= c + 1

    S = max(n_r, n_c)
    K = n_c + 2

    b1r = b1.reshape(1, H)
    b2r = b2.reshape(1, C)

    def row_map(k, s):
        rr = jnp.minimum(k, n_r - 1)
        cc = jnp.where(k >= n_r, n_c - 1, jnp.minimum(s, n_c - 1))
        return (rr, cc)

    def col_map(k, s):
        c = jnp.minimum(jnp.maximum(k - 2, 0), n_c - 1)
        rmax_c = jnp.minimum((BK * (c + 1) - 1) // BI, n_r - 1)
        r = jnp.where(k < 2, 0, jnp.minimum(s, rmax_c))
        return (r, c)

    out = pl.pallas_call(
        _make_body(n_r, n_c, BI, BK, N, TAIL),
        grid=(K, S),
        in_specs=[
            pl.BlockSpec((N, F), lambda k, s: (0, 0)),
            pl.BlockSpec((BI, BK), row_map),
            pl.BlockSpec((BI, BK), col_map),
            pl.BlockSpec((F, H), lambda k, s: (0, 0)),
            pl.BlockSpec((1, H), lambda k, s: (0, 0)),
            pl.BlockSpec((H, C), lambda k, s: (0, 0)),
            pl.BlockSpec((1, C), lambda k, s: (0, 0)),
        ],
        out_specs=pl.BlockSpec((N, C), lambda k, s: (0, 0)),
        out_shape=jax.ShapeDtypeStruct((N, C), jnp.float32),
        scratch_shapes=[
            pltpu.VMEM((n_c * BK, H), jnp.float32),
            pltpu.VMEM((n_c * BK, C), jnp.float32),
            pltpu.VMEM((BI, H), jnp.float32),
        ],
        compiler_params=pltpu.CompilerParams(
            dimension_semantics=("arbitrary", "arbitrary"),
        ),
    )(x, adj, adj, W1, b1r, W2, b2r)

    return out
